# baseline (speedup 1.0000x reference)
import math

import jax
import jax.numpy as jnp
from jax.experimental import pallas as pl

_SQRT_2_OVER_PI = math.sqrt(2.0 / math.pi)

_ROWS = 32768
_COLS = 2048
_BLOCK_ROWS = 512


def _gelu_block(x_ref, o_ref):
    x = x_ref[...]
    x2 = x * x
    u2 = (2.0 * _SQRT_2_OVER_PI) * x + (2.0 * _SQRT_2_OVER_PI * 0.044715) * (x2 * x)
    o_ref[...] = x / (1.0 + jnp.exp(-u2))


def kernel(x, logit_decay, log_tau, log_blend):
    del logit_decay, log_tau, log_blend
    x2 = x.reshape(_ROWS, _COLS)
    out = pl.pallas_call(
        _gelu_block,
        grid=(_ROWS // _BLOCK_ROWS,),
        in_specs=[pl.BlockSpec((_BLOCK_ROWS, _COLS), lambda i: (i, 0))],
        out_specs=pl.BlockSpec((_BLOCK_ROWS, _COLS), lambda i: (i, 0)),
        out_shape=jax.ShapeDtypeStruct((_ROWS, _COLS), x.dtype),
    )(x2)
    return out.reshape(x.shape)


# speedup vs baseline: 1.0498x; 1.0498x over previous
import math

import jax
import jax.numpy as jnp
from jax.experimental import pallas as pl

_SQRT_2_OVER_PI = math.sqrt(2.0 / math.pi)

_ROWS = 32768
_COLS = 2048
_BLOCK_ROWS = 1024


def _gelu_block(x_ref, o_ref):
    x = x_ref[...]
    u = _SQRT_2_OVER_PI * (x + 0.044715 * (x * x * x))
    o_ref[...] = 0.5 * x * (1.0 + jnp.tanh(u))


def kernel(x, logit_decay, log_tau, log_blend):
    del logit_decay, log_tau, log_blend
    x2 = x.reshape(_ROWS, _COLS)
    out = pl.pallas_call(
        _gelu_block,
        grid=(_ROWS // _BLOCK_ROWS,),
        in_specs=[pl.BlockSpec((_BLOCK_ROWS, _COLS), lambda i: (i, 0))],
        out_specs=pl.BlockSpec((_BLOCK_ROWS, _COLS), lambda i: (i, 0)),
        out_shape=jax.ShapeDtypeStruct((_ROWS, _COLS), x.dtype),
    )(x2)
    return out.reshape(x.shape)
